# baseline (speedup 1.0000x reference)
import jax
import jax.numpy as jnp
from jax import lax
from jax.experimental import pallas as pl
from jax.experimental.pallas import tpu as pltpu
from jax.experimental.pallas import tpu_sc as plsc

N = 10000
E = 320000
D = 128

NUM_CORES = 2
NUM_SUBCORES = 16
CPT = 8
NGROUP = D // CPT
EH = E // 2
WR = 250
CHUNKS = EH // WR
VECS = (WR * CPT) // 16
NBUF = 4


def _t2_body(outT0_ref, outT1_ref, input_ref, out_ref):
    out_ref[...] = (input_ref[...] + outT0_ref[...].T) + outT1_ref[...].T


def _merge_out(outT0, outT1, input):
    return pl.pallas_call(
        _t2_body,
        grid=(1,),
        in_specs=[
            pl.BlockSpec((D, N), lambda i: (0, 0)),
            pl.BlockSpec((D, N), lambda i: (0, 0)),
            pl.BlockSpec((N, D), lambda i: (0, 0)),
        ],
        out_specs=pl.BlockSpec((N, D), lambda i: (0, 0)),
        out_shape=jax.ShapeDtypeStruct((N, D), jnp.float32),
    )(outT0, outT1, input)


def _sc_body(idx_hbm, src_hbm, outT0_hbm, outT1_hbm, acc,
             ib0, ib1, ib2, ib3, sb0, sb1, sb2, sb3,
             si0, si1, si2, si3, ss0, ss1, ss2, ss3):
    core = lax.axis_index("c")
    sub = lax.axis_index("s")
    grp = sub % 8
    half = sub // 8
    gg = core * 8 + grp
    col0 = (gg // 2) * (2 * CPT)
    r0 = half * EH

    zeros16 = jnp.zeros((16,), jnp.float32)

    def zero_body(k, _):
        acc[pl.ds(k * 16, 16)] = zeros16
        return 0

    lax.fori_loop(0, (CPT * N) // 16, zero_body, 0, unroll=8)

    lane = lax.iota(jnp.int32, 16)
    cvec = lane % 8
    rpat = lane // 8
    cbase = cvec * N

    def issue(s, ib, sb, isem, ssem):
        row = r0 + s * WR
        pltpu.async_copy(idx_hbm.at[pl.ds(row, WR), pl.ds(col0, 2 * CPT)], ib, isem)
        pltpu.async_copy(src_hbm.at[pl.ds(row, WR), pl.ds(col0, 2 * CPT)], sb, ssem)

    def wait_pair(ib, sb, isem, ssem):
        pltpu.make_async_copy(
            idx_hbm.at[pl.ds(0, WR), pl.ds(col0, 2 * CPT)], ib, isem).wait()
        pltpu.make_async_copy(
            src_hbm.at[pl.ds(0, WR), pl.ds(col0, 2 * CPT)], sb, ssem).wait()

    def compute(ib, sb):
        @plsc.parallel_loop(0, VECS, step=1, unroll=8)
        def _(t):
            rvec = rpat + jnp.full((16,), t * 2, jnp.int32)
            iv = plsc.load_gather(ib, [rvec, cvec])
            sv = plsc.load_gather(sb, [rvec, cvec])
            plsc.addupdate_scatter(acc, [iv + cbase], sv)

    bufs = [(ib0, sb0, si0, ss0), (ib1, sb1, si1, ss1),
            (ib2, sb2, si2, ss2), (ib3, sb3, si3, ss3)]

    for b in range(NBUF):
        issue(jnp.int32(b), *bufs[b])

    def stepn(sn, _):
        s = sn * NBUF
        for b in range(NBUF):
            ib, sb, isem, ssem = bufs[b]
            wait_pair(ib, sb, isem, ssem)
            compute(ib, sb)

            @pl.when(s + b + NBUF < CHUNKS)
            def _():
                issue(s + b + NBUF, ib, sb, isem, ssem)

        return 0

    lax.fori_loop(0, CHUNKS // NBUF, stepn, 0)

    @pl.when(half == 0)
    def _():
        for c in range(CPT):
            pltpu.sync_copy(acc.at[pl.ds(c * N, N)], outT0_hbm.at[col0 + c, :])

    @pl.when(half == 1)
    def _():
        for c in range(CPT):
            pltpu.sync_copy(acc.at[pl.ds(c * N, N)], outT1_hbm.at[col0 + c, :])


def _sc_scatter(index, src):
    mesh = plsc.VectorSubcoreMesh(core_axis_name="c", subcore_axis_name="s")
    f = pl.kernel(
        _sc_body,
        out_type=[
            jax.ShapeDtypeStruct((D, N), jnp.float32),
            jax.ShapeDtypeStruct((D, N), jnp.float32),
        ],
        mesh=mesh,
        scratch_types=[
            pltpu.VMEM((CPT * N,), jnp.float32),
            pltpu.VMEM((WR, 2 * CPT), jnp.int32),
            pltpu.VMEM((WR, 2 * CPT), jnp.int32),
            pltpu.VMEM((WR, 2 * CPT), jnp.int32),
            pltpu.VMEM((WR, 2 * CPT), jnp.int32),
            pltpu.VMEM((WR, 2 * CPT), jnp.float32),
            pltpu.VMEM((WR, 2 * CPT), jnp.float32),
            pltpu.VMEM((WR, 2 * CPT), jnp.float32),
            pltpu.VMEM((WR, 2 * CPT), jnp.float32),
            pltpu.SemaphoreType.DMA,
            pltpu.SemaphoreType.DMA,
            pltpu.SemaphoreType.DMA,
            pltpu.SemaphoreType.DMA,
            pltpu.SemaphoreType.DMA,
            pltpu.SemaphoreType.DMA,
            pltpu.SemaphoreType.DMA,
            pltpu.SemaphoreType.DMA,
        ],
        compiler_params=pltpu.CompilerParams(
            use_tc_tiling_on_sc=False, needs_layout_passes=False
        ),
    )
    return f(index, src)


@jax.jit
def kernel(input, index, src):
    outT0, outT1 = _sc_scatter(index, src)
    return _merge_out(outT0, outT1, input)


# speedup vs baseline: 1.0064x; 1.0064x over previous
import jax
import jax.numpy as jnp
from jax import lax
from jax.experimental import pallas as pl
from jax.experimental.pallas import tpu as pltpu
from jax.experimental.pallas import tpu_sc as plsc

N = 10000
E = 320000
D = 128

NUM_CORES = 2
NUM_SUBCORES = 16
CPT = 8
NGROUP = D // CPT
EH = E // 2
WR = 500
CHUNKS = EH // WR
VECS = (WR * CPT) // 16
NBUF = 4


def _t2_body(outT0_ref, outT1_ref, input_ref, out_ref):
    out_ref[...] = (input_ref[...] + outT0_ref[...].T) + outT1_ref[...].T


def _merge_out(outT0, outT1, input):
    return pl.pallas_call(
        _t2_body,
        grid=(1,),
        in_specs=[
            pl.BlockSpec((D, N), lambda i: (0, 0)),
            pl.BlockSpec((D, N), lambda i: (0, 0)),
            pl.BlockSpec((N, D), lambda i: (0, 0)),
        ],
        out_specs=pl.BlockSpec((N, D), lambda i: (0, 0)),
        out_shape=jax.ShapeDtypeStruct((N, D), jnp.float32),
    )(outT0, outT1, input)


def _sc_body(idx_hbm, src_hbm, outT0_hbm, outT1_hbm, acc,
             ib0, ib1, ib2, ib3, sb0, sb1, sb2, sb3,
             si0, si1, si2, si3, ss0, ss1, ss2, ss3):
    core = lax.axis_index("c")
    sub = lax.axis_index("s")
    grp = sub % 8
    half = sub // 8
    gg = core * 8 + grp
    col0 = gg * CPT
    r0 = half * EH

    zeros16 = jnp.zeros((16,), jnp.float32)

    def zero_body(k, _):
        acc[pl.ds(k * 16, 16)] = zeros16
        return 0

    lax.fori_loop(0, (CPT * N) // 16, zero_body, 0, unroll=8)

    lane = lax.iota(jnp.int32, 16)
    cvec = lane % 8
    rpat = lane // 8
    cbase = cvec * N

    H = WR // 2

    def issue(s, ib, sb, isem, ssem):
        row = r0 + s * WR
        pltpu.async_copy(
            idx_hbm.at[pl.ds(row, H), pl.ds(col0, CPT)], ib.at[pl.ds(0, H)], isem)
        pltpu.async_copy(
            idx_hbm.at[pl.ds(row + H, H), pl.ds(col0, CPT)], ib.at[pl.ds(H, H)], isem)
        pltpu.async_copy(
            src_hbm.at[pl.ds(row, H), pl.ds(col0, CPT)], sb.at[pl.ds(0, H)], ssem)
        pltpu.async_copy(
            src_hbm.at[pl.ds(row + H, H), pl.ds(col0, CPT)], sb.at[pl.ds(H, H)], ssem)

    def wait_pair(ib, sb, isem, ssem):
        pltpu.make_async_copy(
            idx_hbm.at[pl.ds(0, WR), pl.ds(col0, CPT)], ib, isem).wait()
        pltpu.make_async_copy(
            src_hbm.at[pl.ds(0, WR), pl.ds(col0, CPT)], sb, ssem).wait()

    def compute(ib, sb):
        @plsc.parallel_loop(0, VECS, step=1, unroll=8)
        def _(t):
            rvec = rpat + jnp.full((16,), t * 2, jnp.int32)
            iv = plsc.load_gather(ib, [rvec, cvec])
            sv = plsc.load_gather(sb, [rvec, cvec])
            plsc.addupdate_scatter(acc, [iv + cbase], sv)

    bufs = [(ib0, sb0, si0, ss0), (ib1, sb1, si1, ss1),
            (ib2, sb2, si2, ss2), (ib3, sb3, si3, ss3)]

    for b in range(NBUF):
        issue(jnp.int32(b), *bufs[b])

    def stepn(sn, _):
        s = sn * NBUF
        for b in range(NBUF):
            ib, sb, isem, ssem = bufs[b]
            wait_pair(ib, sb, isem, ssem)
            compute(ib, sb)

            @pl.when(s + b + NBUF < CHUNKS)
            def _():
                issue(s + b + NBUF, ib, sb, isem, ssem)

        return 0

    lax.fori_loop(0, CHUNKS // NBUF, stepn, 0)

    @pl.when(half == 0)
    def _():
        for c in range(CPT):
            pltpu.sync_copy(acc.at[pl.ds(c * N, N)], outT0_hbm.at[col0 + c, :])

    @pl.when(half == 1)
    def _():
        for c in range(CPT):
            pltpu.sync_copy(acc.at[pl.ds(c * N, N)], outT1_hbm.at[col0 + c, :])


def _sc_scatter(index, src):
    mesh = plsc.VectorSubcoreMesh(core_axis_name="c", subcore_axis_name="s")
    f = pl.kernel(
        _sc_body,
        out_type=[
            jax.ShapeDtypeStruct((D, N), jnp.float32),
            jax.ShapeDtypeStruct((D, N), jnp.float32),
        ],
        mesh=mesh,
        scratch_types=[
            pltpu.VMEM((CPT * N,), jnp.float32),
            pltpu.VMEM((WR, CPT), jnp.int32),
            pltpu.VMEM((WR, CPT), jnp.int32),
            pltpu.VMEM((WR, CPT), jnp.int32),
            pltpu.VMEM((WR, CPT), jnp.int32),
            pltpu.VMEM((WR, CPT), jnp.float32),
            pltpu.VMEM((WR, CPT), jnp.float32),
            pltpu.VMEM((WR, CPT), jnp.float32),
            pltpu.VMEM((WR, CPT), jnp.float32),
            pltpu.SemaphoreType.DMA,
            pltpu.SemaphoreType.DMA,
            pltpu.SemaphoreType.DMA,
            pltpu.SemaphoreType.DMA,
            pltpu.SemaphoreType.DMA,
            pltpu.SemaphoreType.DMA,
            pltpu.SemaphoreType.DMA,
            pltpu.SemaphoreType.DMA,
        ],
        compiler_params=pltpu.CompilerParams(
            use_tc_tiling_on_sc=False, needs_layout_passes=False
        ),
    )
    return f(index, src)


@jax.jit
def kernel(input, index, src):
    outT0, outT1 = _sc_scatter(index, src)
    return _merge_out(outT0, outT1, input)


# speedup vs baseline: 1.0072x; 1.0008x over previous
import jax
import jax.numpy as jnp
from jax import lax
from jax.experimental import pallas as pl
from jax.experimental.pallas import tpu as pltpu
from jax.experimental.pallas import tpu_sc as plsc

N = 10000
E = 320000
D = 128

NUM_CORES = 2
NUM_SUBCORES = 16
CPT = 8
NGROUP = D // CPT
EH = E // 2
WR = 500
CHUNKS = EH // WR
VECS = (WR * CPT) // 16
NBUF = 4


def _t2_body(outT0_ref, outT1_ref, input_ref, out_ref):
    out_ref[...] = (input_ref[...] + outT0_ref[...].T) + outT1_ref[...].T


def _merge_out(outT0, outT1, input):
    return pl.pallas_call(
        _t2_body,
        grid=(1,),
        in_specs=[
            pl.BlockSpec((D, N), lambda i: (0, 0)),
            pl.BlockSpec((D, N), lambda i: (0, 0)),
            pl.BlockSpec((N, D), lambda i: (0, 0)),
        ],
        out_specs=pl.BlockSpec((N, D), lambda i: (0, 0)),
        out_shape=jax.ShapeDtypeStruct((N, D), jnp.float32),
    )(outT0, outT1, input)


def _sc_body(idx_hbm, src_hbm, outT0_hbm, outT1_hbm, acc,
             ib0, ib1, ib2, ib3, sb0, sb1, sb2, sb3,
             si0, si1, si2, si3, ss0, ss1, ss2, ss3):
    core = lax.axis_index("c")
    sub = lax.axis_index("s")
    grp = sub % 8
    half = sub // 8
    gg = core * 8 + grp
    col0 = gg * CPT
    r0 = half * EH

    zeros16 = jnp.zeros((16,), jnp.float32)

    def zero_body(k, _):
        acc[pl.ds(k * 16, 16)] = zeros16
        return 0

    lax.fori_loop(0, (CPT * N) // 16, zero_body, 0, unroll=8)

    lane = lax.iota(jnp.int32, 16)
    cvec = lane % 8
    rpat = lane // 8
    cbase = cvec * N

    def issue(s, ib, sb, isem, ssem):
        row = r0 + s * WR
        pltpu.async_copy(idx_hbm.at[pl.ds(row, WR), pl.ds(col0, CPT)], ib, isem)
        pltpu.async_copy(src_hbm.at[pl.ds(row, WR), pl.ds(col0, CPT)], sb, ssem)

    def wait_pair(ib, sb, isem, ssem):
        pltpu.make_async_copy(
            idx_hbm.at[pl.ds(0, WR), pl.ds(col0, CPT)], ib, isem).wait()
        pltpu.make_async_copy(
            src_hbm.at[pl.ds(0, WR), pl.ds(col0, CPT)], sb, ssem).wait()

    def compute(ib, sb):
        @plsc.parallel_loop(0, VECS, step=1, unroll=8)
        def _(t):
            rvec = rpat + jnp.full((16,), t * 2, jnp.int32)
            iv = plsc.load_gather(ib, [rvec, cvec])
            sv = plsc.load_gather(sb, [rvec, cvec])
            plsc.addupdate_scatter(acc, [iv + cbase], sv)

    bufs = [(ib0, sb0, si0, ss0), (ib1, sb1, si1, ss1),
            (ib2, sb2, si2, ss2), (ib3, sb3, si3, ss3)]

    for b in range(NBUF):
        issue(jnp.int32(b), *bufs[b])

    def stepn(sn, _):
        s = sn * NBUF
        for b in range(NBUF):
            ib, sb, isem, ssem = bufs[b]
            wait_pair(ib, sb, isem, ssem)
            compute(ib, sb)

            @pl.when(s + b + NBUF < CHUNKS)
            def _():
                issue(s + b + NBUF, ib, sb, isem, ssem)

        return 0

    lax.fori_loop(0, CHUNKS // NBUF, stepn, 0)

    @pl.when(half == 0)
    def _():
        for c in range(CPT):
            pltpu.sync_copy(acc.at[pl.ds(c * N, N)], outT0_hbm.at[col0 + c, :])

    @pl.when(half == 1)
    def _():
        for c in range(CPT):
            pltpu.sync_copy(acc.at[pl.ds(c * N, N)], outT1_hbm.at[col0 + c, :])


def _sc_scatter(index, src):
    mesh = plsc.VectorSubcoreMesh(core_axis_name="c", subcore_axis_name="s")
    f = pl.kernel(
        _sc_body,
        out_type=[
            jax.ShapeDtypeStruct((D, N), jnp.float32),
            jax.ShapeDtypeStruct((D, N), jnp.float32),
        ],
        mesh=mesh,
        scratch_types=[
            pltpu.VMEM((CPT * N,), jnp.float32),
            pltpu.VMEM((WR, CPT), jnp.int32),
            pltpu.VMEM((WR, CPT), jnp.int32),
            pltpu.VMEM((WR, CPT), jnp.int32),
            pltpu.VMEM((WR, CPT), jnp.int32),
            pltpu.VMEM((WR, CPT), jnp.float32),
            pltpu.VMEM((WR, CPT), jnp.float32),
            pltpu.VMEM((WR, CPT), jnp.float32),
            pltpu.VMEM((WR, CPT), jnp.float32),
            pltpu.SemaphoreType.DMA,
            pltpu.SemaphoreType.DMA,
            pltpu.SemaphoreType.DMA,
            pltpu.SemaphoreType.DMA,
            pltpu.SemaphoreType.DMA,
            pltpu.SemaphoreType.DMA,
            pltpu.SemaphoreType.DMA,
            pltpu.SemaphoreType.DMA,
        ],
        compiler_params=pltpu.CompilerParams(
            use_tc_tiling_on_sc=False, needs_layout_passes=False
        ),
    )
    return f(index, src)


@jax.jit
def kernel(input, index, src):
    outT0, outT1 = _sc_scatter(index, src)
    return _merge_out(outT0, outT1, input)
